# baseline (speedup 1.0000x reference)
import functools

import jax
import jax.numpy as jnp
from jax import lax
from jax.experimental import pallas as pl
from jax.experimental.pallas import tpu as pltpu
from jax.experimental.pallas import tpu_sc as plsc

L = 2048
NCORES = 2
NSUB = 16
NW = NCORES * NSUB
ROWS_PER_W = L // NW
BLK = 8
NBLK = ROWS_PER_W // BLK
NBUF = 3
LANES = 16
NCH = L // LANES
TB = 128

PARTNER_LUT = 0x4A52


def _body(con_hbm, feat_hbm, out_hbm, seq_v, codes_v, rowp_v, blk_v,
          in_v, out_v, sh_seq, sem_in, sem_out):
    cid = lax.axis_index("c")
    sid = lax.axis_index("s")
    wid = sid * NCORES + cid
    row0 = wid * ROWS_PER_W

    def in_copy(blk, buf):
        return pltpu.make_async_copy(
            con_hbm.at[0, 0, pl.ds(row0 + blk * BLK, BLK)], in_v.at[buf],
            sem_in)

    def out_copy(blk, buf):
        return pltpu.make_async_copy(
            out_v.at[buf], out_hbm.at[0, 0, pl.ds(row0 + blk * BLK, BLK)],
            sem_out)

    for k in range(NBUF):
        in_copy(k, k).start()

    c = lax.rem(sid, 4)
    q = lax.div(sid, 4)
    for j in range(4):
        r0 = q * 512 + j * TB
        pltpu.sync_copy(feat_hbm.at[0, c, pl.ds(r0, TB), pl.ds(0, TB)], blk_v)
        pltpu.sync_copy(blk_v.at[:, 0], sh_seq.at[c, pl.ds(r0, TB)])
    plsc.subcore_barrier()
    pltpu.sync_copy(sh_seq, seq_v)

    @plsc.parallel_loop(0, NCH, unroll=2)
    def _class_chunk(i):
        sl = pl.ds(i * LANES, LANES)
        v = seq_v[0, sl]
        cc = jnp.zeros((LANES,), jnp.int32)
        for k in (1, 2, 3):
            s = seq_v[k, sl]
            upd = s > v
            cc = jnp.where(upd, k, cc)
            v = jnp.maximum(v, s)
        codes_v[sl] = jnp.int32(1) << cc
        rowp_v[sl] = (jnp.int32(PARTNER_LUT) >> (cc * 4)) & 0xF

    idx16 = lax.iota(jnp.int32, LANES)
    ftab = jnp.where(idx16 == 0, 0.0, 1.0).astype(jnp.float32)

    def block(blk, _):
        b = lax.rem(blk, NBUF)
        in_copy(blk, b).wait()

        @pl.when(blk >= NBUF)
        def _():
            out_copy(blk - NBUF, b).wait()

        rowp16 = rowp_v[pl.ds(row0 + blk * BLK, LANES)]
        pvecs = [
            rowp16.at[jnp.full((LANES,), rr, jnp.int32)].get(
                mode="promise_in_bounds")
            for rr in range(BLK)
        ]

        @plsc.parallel_loop(0, NCH, unroll=2)
        def _cols(ci):
            sl = pl.ds(ci * LANES, LANES)
            code = codes_v[sl]
            for rr in range(BLK):
                x = code & pvecs[rr]
                fm = ftab.at[x].get(mode="promise_in_bounds")
                out_v[b, rr, sl] = in_v[b, rr, sl] * fm

        out_copy(blk, b).start()

        @pl.when(blk + NBUF < NBLK)
        def _():
            in_copy(blk + NBUF, b).start()

        return 0

    lax.fori_loop(0, NBLK, block, 0)
    for k in range(NBUF):
        blk = NBLK - NBUF + k
        out_copy(blk, lax.rem(jnp.int32(blk), NBUF)).wait()


@jax.jit
def _canonicalize(con, feat):
    mesh = plsc.VectorSubcoreMesh(core_axis_name="c", subcore_axis_name="s")
    f = functools.partial(
        pl.kernel,
        mesh=mesh,
        out_type=jax.ShapeDtypeStruct((1, 1, L, L), jnp.float32),
        scratch_types=[
            pltpu.VMEM((4, L), jnp.float32),
            pltpu.VMEM((L,), jnp.int32),
            pltpu.VMEM((L + LANES,), jnp.int32),
            pltpu.VMEM((TB, TB), jnp.float32),
            pltpu.VMEM((NBUF, BLK, L), jnp.float32),
            pltpu.VMEM((NBUF, BLK, L), jnp.float32),
            pltpu.VMEM_SHARED((4, L), jnp.float32),
            pltpu.SemaphoreType.DMA,
            pltpu.SemaphoreType.DMA,
        ],
    )(_body)
    return f(con, feat)


def kernel(con, feat):
    return _canonicalize(con, feat)


# speedup vs baseline: 1.1127x; 1.1127x over previous
import functools

import jax
import jax.numpy as jnp
from jax import lax
from jax.experimental import pallas as pl
from jax.experimental.pallas import tpu as pltpu
from jax.experimental.pallas import tpu_sc as plsc

L = 2048
NCORES = 2
NSUB = 16
NW = NCORES * NSUB
ROWS_PER_W = L // NW
BLK = 8
NBLK = ROWS_PER_W // BLK
NBUF = 3
LANES = 16
NCH = L // LANES

PARTNER_LUT = 0x4A52


def _body(con_hbm, seq_hbm, out_hbm, seq_v, codes_v, rowp_v, in_v, out_v,
          sem_in, sem_out):
    wid = lax.axis_index("s") * NCORES + lax.axis_index("c")
    row0 = wid * ROWS_PER_W

    def in_copy(blk, buf):
        return pltpu.make_async_copy(
            con_hbm.at[pl.ds(row0 + blk * BLK, BLK)], in_v.at[buf], sem_in)

    def out_copy(blk, buf):
        return pltpu.make_async_copy(
            out_v.at[buf], out_hbm.at[pl.ds(row0 + blk * BLK, BLK)], sem_out)

    for k in range(NBUF):
        in_copy(k, k).start()

    pltpu.sync_copy(seq_hbm, seq_v)

    @plsc.parallel_loop(0, NCH, unroll=2)
    def _class_chunk(i):
        sl = pl.ds(i * LANES, LANES)
        v = seq_v[0, sl]
        c = jnp.zeros((LANES,), jnp.int32)
        for k in (1, 2, 3):
            s = seq_v[k, sl]
            upd = s > v
            c = jnp.where(upd, k, c)
            v = jnp.maximum(v, s)
        codes_v[sl] = jnp.int32(1) << c
        rowp_v[sl] = (jnp.int32(PARTNER_LUT) >> (c * 4)) & 0xF

    idx16 = lax.iota(jnp.int32, LANES)
    ftab = jnp.where(idx16 == 0, 0.0, 1.0).astype(jnp.float32)

    def block(blk, _):
        b = lax.rem(blk, NBUF)
        in_copy(blk, b).wait()

        @pl.when(blk >= NBUF)
        def _():
            out_copy(blk - NBUF, b).wait()

        rowp16 = rowp_v[pl.ds(row0 + blk * BLK, LANES)]
        pvecs = [
            rowp16.at[jnp.full((LANES,), rr, jnp.int32)].get(
                mode="promise_in_bounds")
            for rr in range(BLK)
        ]

        @plsc.parallel_loop(0, NCH, unroll=2)
        def _cols(ci):
            sl = pl.ds(ci * LANES, LANES)
            code = codes_v[sl]
            for rr in range(BLK):
                x = code & pvecs[rr]
                fm = ftab.at[x].get(mode="promise_in_bounds")
                out_v[b, rr, sl] = in_v[b, rr, sl] * fm

        out_copy(blk, b).start()

        @pl.when(blk + NBUF < NBLK)
        def _():
            in_copy(blk + NBUF, b).start()

        return 0

    lax.fori_loop(0, NBLK, block, 0)
    for k in range(NBUF):
        blk = NBLK - NBUF + k
        out_copy(blk, lax.rem(jnp.int32(blk), NBUF)).wait()


@jax.jit
def _canonicalize(con2d, seq):
    mesh = plsc.VectorSubcoreMesh(core_axis_name="c", subcore_axis_name="s")
    f = functools.partial(
        pl.kernel,
        mesh=mesh,
        out_type=jax.ShapeDtypeStruct((L, L), jnp.float32),
        scratch_types=[
            pltpu.VMEM((4, L), jnp.float32),
            pltpu.VMEM((L,), jnp.int32),
            pltpu.VMEM((L + LANES,), jnp.int32),
            pltpu.VMEM((NBUF, BLK, L), jnp.float32),
            pltpu.VMEM((NBUF, BLK, L), jnp.float32),
            pltpu.SemaphoreType.DMA,
            pltpu.SemaphoreType.DMA,
        ],
    )(_body)
    return f(con2d, seq)


def kernel(con, feat):
    con2d = con.reshape(L, L)
    seq = feat[0, :4, :, 0]
    out = _canonicalize(con2d, seq)
    return out.reshape(con.shape)


# speedup vs baseline: 1.1298x; 1.0153x over previous
import functools

import jax
import jax.numpy as jnp
from jax import lax
from jax.experimental import pallas as pl
from jax.experimental.pallas import tpu as pltpu
from jax.experimental.pallas import tpu_sc as plsc

L = 2048
NCORES = 2
NSUB = 16
NW = NCORES * NSUB
ROWS_PER_W = L // NW
BLK = 8
NBLK = ROWS_PER_W // BLK
NBUF_I = 4
NBUF_O = 2
LANES = 16
NCH = L // LANES

PARTNER_LUT = 0x4A52


def _body(con_hbm, seq_hbm, out_hbm, seq_v, codes_v, rowp_v, in_v, out_v,
          sem_in, sem_out):
    wid = lax.axis_index("s") * NCORES + lax.axis_index("c")
    row0 = wid * ROWS_PER_W

    def in_copy(blk, buf):
        return pltpu.make_async_copy(
            con_hbm.at[pl.ds(row0 + blk * BLK, BLK)], in_v.at[buf], sem_in)

    def out_copy(blk, buf):
        return pltpu.make_async_copy(
            out_v.at[buf], out_hbm.at[pl.ds(row0 + blk * BLK, BLK)], sem_out)

    for k in range(NBUF_I - 1):
        in_copy(k, k).start()

    pltpu.sync_copy(seq_hbm, seq_v)

    @plsc.parallel_loop(0, NCH, unroll=2)
    def _class_chunk(i):
        sl = pl.ds(i * LANES, LANES)
        v = seq_v[0, sl]
        c = jnp.zeros((LANES,), jnp.int32)
        for k in (1, 2, 3):
            s = seq_v[k, sl]
            upd = s > v
            c = jnp.where(upd, k, c)
            v = jnp.maximum(v, s)
        codes_v[sl] = jnp.int32(1) << c
        rowp_v[sl] = (jnp.int32(PARTNER_LUT) >> (c * 4)) & 0xF

    idx16 = lax.iota(jnp.int32, LANES)
    ftab = jnp.where(idx16 == 0, 0.0, 1.0).astype(jnp.float32)

    def block(blk, _):
        b = lax.rem(blk, NBUF_I)
        bo = lax.rem(blk, NBUF_O)

        @pl.when(blk + NBUF_I - 1 < NBLK)
        def _():
            in_copy(blk + NBUF_I - 1, lax.rem(blk + NBUF_I - 1, NBUF_I)).start()

        in_copy(blk, b).wait()

        @pl.when(blk >= NBUF_O)
        def _():
            out_copy(blk - NBUF_O, bo).wait()

        rowp16 = rowp_v[pl.ds(row0 + blk * BLK, LANES)]
        pvecs = [
            rowp16.at[jnp.full((LANES,), rr, jnp.int32)].get(
                mode="promise_in_bounds")
            for rr in range(BLK)
        ]

        @plsc.parallel_loop(0, NCH, unroll=2)
        def _cols(ci):
            sl = pl.ds(ci * LANES, LANES)
            code = codes_v[sl]
            for rr in range(BLK):
                x = code & pvecs[rr]
                fm = ftab.at[x].get(mode="promise_in_bounds")
                out_v[bo, rr, sl] = in_v[b, rr, sl] * fm

        out_copy(blk, bo).start()
        return 0

    lax.fori_loop(0, NBLK, block, 0)
    for k in range(NBUF_O):
        blk = NBLK - NBUF_O + k
        out_copy(blk, lax.rem(jnp.int32(blk), NBUF_O)).wait()


@jax.jit
def _canonicalize(con2d, seq):
    mesh = plsc.VectorSubcoreMesh(core_axis_name="c", subcore_axis_name="s")
    f = functools.partial(
        pl.kernel,
        mesh=mesh,
        out_type=jax.ShapeDtypeStruct((L, L), jnp.float32),
        scratch_types=[
            pltpu.VMEM((4, L), jnp.float32),
            pltpu.VMEM((L,), jnp.int32),
            pltpu.VMEM((L + LANES,), jnp.int32),
            pltpu.VMEM((NBUF_I, BLK, L), jnp.float32),
            pltpu.VMEM((NBUF_O, BLK, L), jnp.float32),
            pltpu.SemaphoreType.DMA,
            pltpu.SemaphoreType.DMA,
        ],
    )(_body)
    return f(con2d, seq)


def kernel(con, feat):
    con2d = con.reshape(L, L)
    seq = feat[0, :4, :, 0]
    out = _canonicalize(con2d, seq)
    return out.reshape(con.shape)
